# baseline (speedup 1.0000x reference)
import functools
from math import pi, log

import jax
import jax.numpy as jnp
from jax.experimental import pallas as pl

N = 10000
E = 320000
EH = 320000
H = 128
NF = 128
NG = 25
NB = 512
L = 3
CUT = 5.0

_LOG2 = log(2.0)


def _ssp(x):
    return jnp.maximum(x, 0.0) + jnp.log(1.0 + jnp.exp(-jnp.abs(x))) - _LOG2


def _edge_filter_body(dist_ref, fh_ref, w1_ref, b1_ref, w2_ref, b2_ref,
                      mw1_ref, mb1_ref, mw2_ref, mb2_ref, wout_ref, whout_ref):
    d = dist_ref[...]
    spacing = CUT / (NG - 1)
    coeff = -0.5 / (spacing * spacing)
    offs = jax.lax.broadcasted_iota(jnp.int32, (1, NG), 1).astype(jnp.float32) * spacing
    de = jnp.exp(coeff * (d - offs) ** 2)
    h1 = _ssp(jnp.dot(de, w1_ref[0], preferred_element_type=jnp.float32)
              + b1_ref[0])
    w = jnp.dot(h1, w2_ref[0], preferred_element_type=jnp.float32) + b2_ref[0]
    c = 0.5 * (jnp.cos(d * (pi / CUT)) + 1.0)
    wout_ref[...] = (w * c)[None]

    fh = fh_ref[...]
    hh = _ssp(jnp.dot(fh, mw1_ref[0], preferred_element_type=jnp.float32)
              + mb1_ref[0])
    wh = jnp.dot(hh, mw2_ref[0], preferred_element_type=jnp.float32) + mb2_ref[0]
    whout_ref[...] = wh[None]


def _edge_filter(dist, fea_hull, params, block_e=3200):
    ne = E // block_e
    grid = (L, ne)
    d2 = dist.reshape(E, 1)
    out_shape = [
        jax.ShapeDtypeStruct((L, E, NF), jnp.float32),
        jax.ShapeDtypeStruct((L, EH, NF), jnp.float32),
    ]
    in_specs = [
        pl.BlockSpec((block_e, 1), lambda l, e: (e, 0)),
        pl.BlockSpec((block_e, 7), lambda l, e: (e, 0)),
        pl.BlockSpec((1, NG, NF), lambda l, e: (l, 0, 0)),
        pl.BlockSpec((1, 1, NF), lambda l, e: (l, 0, 0)),
        pl.BlockSpec((1, NF, NF), lambda l, e: (l, 0, 0)),
        pl.BlockSpec((1, 1, NF), lambda l, e: (l, 0, 0)),
        pl.BlockSpec((1, 7, NF), lambda l, e: (l, 0, 0)),
        pl.BlockSpec((1, 1, NF), lambda l, e: (l, 0, 0)),
        pl.BlockSpec((1, NF, NF), lambda l, e: (l, 0, 0)),
        pl.BlockSpec((1, 1, NF), lambda l, e: (l, 0, 0)),
    ]
    out_specs = [
        pl.BlockSpec((1, block_e, NF), lambda l, e: (l, e, 0)),
        pl.BlockSpec((1, block_e, NF), lambda l, e: (l, e, 0)),
    ]
    return pl.pallas_call(
        _edge_filter_body,
        grid=grid,
        in_specs=in_specs,
        out_specs=out_specs,
        out_shape=out_shape,
    )(d2, fea_hull,
      params["ue_mlp_W1"], params["ue_mlp_b1"].reshape(L, 1, NF),
      params["ue_mlp_W2"], params["ue_mlp_b2"].reshape(L, 1, NF),
      params["ue_mh_W1"], params["ue_mh_b1"].reshape(L, 1, NF),
      params["ue_mh_W2"], params["ue_mh_b2"].reshape(L, 1, NF))


def kernel(z, dist, edge_index, fea_hull, edge_index_hull, batch, params):
    w_all, wh_all = _edge_filter(dist, fea_hull, params)

    v = params["emb_table"][z]
    src, dst = edge_index[0], edge_index[1]
    srch, dsth = edge_index_hull[0], edge_index_hull[1]

    for l in range(L):
        e = (v @ params["ue_lin"][l])[src] * w_all[l]
        eh = (v @ params["ue_lin_hull"][l])[srch] * wh_all[l]
        out = jax.ops.segment_sum(e, dst, num_segments=N)
        out = _ssp(out @ params["uv_lin1_W"][l] + params["uv_lin1_b"][l]) \
            @ params["uv_lin2_W"][l] + params["uv_lin2_b"][l]
        outh = jax.ops.segment_sum(eh, dsth, num_segments=N)
        outh = _ssp(outh @ params["uv_lin1h_W"][l] + params["uv_lin1h_b"][l]) \
            @ params["uv_lin2h_W"][l] + params["uv_lin2h_b"][l]
        out = _ssp(jnp.concatenate([out, outh], axis=1) @ params["uv_cat_W"][l]
                   + params["uv_cat_b"][l])
        v = v + out

    h = _ssp(v @ params["uu_lin1_W"] + params["uu_lin1_b"])
    o = h @ params["uu_lin2_W"] + params["uu_lin2_b"]
    return jax.ops.segment_sum(o, batch, num_segments=NB)


# speedup vs baseline: 1.9087x; 1.9087x over previous
import functools
from math import pi, log

import jax
import jax.numpy as jnp
from jax import lax
from jax.experimental import pallas as pl
from jax.experimental.pallas import tpu as pltpu
from jax.experimental.pallas import tpu_sc as plsc

N = 10000
E = 320000
EH = 320000
H = 128
NF = 128
NG = 25
NB = 512
L = 3
CUT = 5.0

_LOG2 = log(2.0)


def _ssp(x):
    return jnp.maximum(x, 0.0) + jnp.log(1.0 + jnp.exp(-jnp.abs(x))) - _LOG2


def _edge_filter_body(dist_ref, fh_ref, w1_ref, b1_ref, w2_ref, b2_ref,
                      mw1_ref, mb1_ref, mw2_ref, mb2_ref, wout_ref, whout_ref):
    d = dist_ref[...]
    spacing = CUT / (NG - 1)
    coeff = -0.5 / (spacing * spacing)
    offs = jax.lax.broadcasted_iota(jnp.int32, (1, NG), 1).astype(jnp.float32) * spacing
    de = jnp.exp(coeff * (d - offs) ** 2)
    h1 = _ssp(jnp.dot(de, w1_ref[0], preferred_element_type=jnp.float32)
              + b1_ref[0])
    w = jnp.dot(h1, w2_ref[0], preferred_element_type=jnp.float32) + b2_ref[0]
    c = 0.5 * (jnp.cos(d * (pi / CUT)) + 1.0)
    wout_ref[...] = (w * c)[None]

    fh = fh_ref[...]
    hh = _ssp(jnp.dot(fh, mw1_ref[0], preferred_element_type=jnp.float32)
              + mb1_ref[0])
    wh = jnp.dot(hh, mw2_ref[0], preferred_element_type=jnp.float32) + mb2_ref[0]
    whout_ref[...] = wh[None]


def _edge_filter(dist, fea_hull, params, block_e=3200):
    ne = E // block_e
    grid = (L, ne)
    d2 = dist.reshape(E, 1)
    out_shape = [
        jax.ShapeDtypeStruct((L, E, NF), jnp.float32),
        jax.ShapeDtypeStruct((L, EH, NF), jnp.float32),
    ]
    in_specs = [
        pl.BlockSpec((block_e, 1), lambda l, e: (e, 0)),
        pl.BlockSpec((block_e, 7), lambda l, e: (e, 0)),
        pl.BlockSpec((1, NG, NF), lambda l, e: (l, 0, 0)),
        pl.BlockSpec((1, 1, NF), lambda l, e: (l, 0, 0)),
        pl.BlockSpec((1, NF, NF), lambda l, e: (l, 0, 0)),
        pl.BlockSpec((1, 1, NF), lambda l, e: (l, 0, 0)),
        pl.BlockSpec((1, 7, NF), lambda l, e: (l, 0, 0)),
        pl.BlockSpec((1, 1, NF), lambda l, e: (l, 0, 0)),
        pl.BlockSpec((1, NF, NF), lambda l, e: (l, 0, 0)),
        pl.BlockSpec((1, 1, NF), lambda l, e: (l, 0, 0)),
    ]
    out_specs = [
        pl.BlockSpec((1, block_e, NF), lambda l, e: (l, e, 0)),
        pl.BlockSpec((1, block_e, NF), lambda l, e: (l, e, 0)),
    ]
    return pl.pallas_call(
        _edge_filter_body,
        grid=grid,
        in_specs=in_specs,
        out_specs=out_specs,
        out_shape=out_shape,
    )(d2, fea_hull,
      params["ue_mlp_W1"], params["ue_mlp_b1"].reshape(L, 1, NF),
      params["ue_mlp_W2"], params["ue_mlp_b2"].reshape(L, 1, NF),
      params["ue_mh_W1"], params["ue_mh_b1"].reshape(L, 1, NF),
      params["ue_mh_W2"], params["ue_mh_b2"].reshape(L, 1, NF))


_NC = 2
_NS = 16
_NW = _NC * _NS
_CH = 128


def _seg_body(zeros_hbm, vl_hbm, w_hbm, src_hbm, dst_hbm, out_hbm,
              sidx, didx, gbuf, wbuf, acc, sem):
    c = lax.axis_index("c")
    s = lax.axis_index("s")
    wid = s * _NC + c

    @pl.when(s == 0)
    def _():
        pltpu.sync_copy(zeros_hbm, acc)

    plsc.subcore_barrier()

    nchunks = E // _CH
    lo = wid * nchunks // _NW
    hi = (wid + 1) * nchunks // _NW

    def chunk(k, carry):
        base = pl.multiple_of(k * _CH, _CH)
        pltpu.sync_copy(src_hbm.at[pl.ds(base, _CH)], sidx)
        pltpu.sync_copy(dst_hbm.at[pl.ds(base, _CH)], didx)
        pltpu.async_copy(vl_hbm.at[sidx], gbuf, sem).wait()
        pltpu.sync_copy(w_hbm.at[pl.ds(base, _CH)], wbuf)

        def row(i, carry2):
            for j in range(NF // 16):
                g = gbuf[i, pl.ds(j * 16, 16)]
                w = wbuf[i, pl.ds(j * 16, 16)]
                gbuf[i, pl.ds(j * 16, 16)] = g * w
            return carry2

        lax.fori_loop(0, _CH, row, 0, unroll=False)
        pltpu.sync_copy(gbuf, acc.at[didx], add=True)
        return carry

    lax.fori_loop(lo, hi, chunk, 0, unroll=False)
    plsc.subcore_barrier()

    rows = 624
    base_r = pl.multiple_of(s * rows, 8)

    @pl.when(s < _NS - 1)
    def _():
        pltpu.sync_copy(acc.at[pl.ds(base_r, rows)],
                        out_hbm.at[c, pl.ds(base_r, rows)])

    @pl.when(s == _NS - 1)
    def _():
        last = N - (_NS - 1) * rows
        pltpu.sync_copy(acc.at[pl.ds((_NS - 1) * rows, last)],
                        out_hbm.at[c, pl.ds((_NS - 1) * rows, last)])


@functools.partial(jax.jit, static_argnames=())
def _sc_gather_mul_scatter(zeros, vl, w, src, dst):
    mesh = plsc.VectorSubcoreMesh(core_axis_name="c", subcore_axis_name="s",
                                  num_cores=_NC, num_subcores=_NS)
    f = pl.kernel(
        _seg_body,
        out_type=jax.ShapeDtypeStruct((_NC, N, NF), jnp.float32),
        mesh=mesh,
        scratch_types=[
            pltpu.VMEM((_CH,), jnp.int32),
            pltpu.VMEM((_CH,), jnp.int32),
            pltpu.VMEM((_CH, NF), jnp.float32),
            pltpu.VMEM((_CH, NF), jnp.float32),
            pltpu.VMEM_SHARED((N, NF), jnp.float32),
            pltpu.SemaphoreType.DMA,
        ],
    )
    return f(zeros, vl, w, src, dst)


def kernel(z, dist, edge_index, fea_hull, edge_index_hull, batch, params):
    w_all, wh_all = _edge_filter(dist, fea_hull, params)

    v = params["emb_table"][z]
    src, dst = edge_index[0], edge_index[1]
    srch, dsth = edge_index_hull[0], edge_index_hull[1]

    zeros = jnp.zeros((N, NF), jnp.float32)
    for l in range(L):
        vl = v @ params["ue_lin"][l]
        vlh = v @ params["ue_lin_hull"][l]
        out_p = _sc_gather_mul_scatter(zeros, vl, w_all[l], src, dst)
        out = out_p[0] + out_p[1]
        outh_p = _sc_gather_mul_scatter(zeros, vlh, wh_all[l], srch, dsth)
        outh = outh_p[0] + outh_p[1]
        out = _ssp(out @ params["uv_lin1_W"][l] + params["uv_lin1_b"][l]) \
            @ params["uv_lin2_W"][l] + params["uv_lin2_b"][l]
        outh = _ssp(outh @ params["uv_lin1h_W"][l] + params["uv_lin1h_b"][l]) \
            @ params["uv_lin2h_W"][l] + params["uv_lin2h_b"][l]
        out = _ssp(jnp.concatenate([out, outh], axis=1) @ params["uv_cat_W"][l]
                   + params["uv_cat_b"][l])
        v = v + out

    h = _ssp(v @ params["uu_lin1_W"] + params["uu_lin1_b"])
    o = h @ params["uu_lin2_W"] + params["uu_lin2_b"]
    return jax.ops.segment_sum(o, batch, num_segments=NB)


# speedup vs baseline: 2.5885x; 1.3561x over previous
import functools
from math import pi, log

import numpy as np
import jax
import jax.numpy as jnp
from jax import lax
from jax.experimental import pallas as pl
from jax.experimental.pallas import tpu as pltpu
from jax.experimental.pallas import tpu_sc as plsc

N = 10000
E = 320000
EH = 320000
H = 128
NF = 128
NG = 25
NB = 512
L = 3
CUT = 5.0

_LOG2 = log(2.0)


def _ssp(x):
    return jnp.maximum(x, 0.0) + jnp.log(1.0 + jnp.exp(-jnp.abs(x))) - _LOG2


_SPACING = CUT / (NG - 1)
_COEFF = -0.5 / (_SPACING * _SPACING)
_OFFS = np.arange(NG, dtype=np.float32) * _SPACING
_M4 = np.zeros((4, NG), dtype=np.float32)
_M4[0] = _COEFF
_M4[1] = -2.0 * _COEFF * _OFFS
_M4[2] = _COEFF * _OFFS * _OFFS


def _edge_filter_body(d_ref, c_ref, fh_ref, w1_ref, b1_ref, w2_ref, b2_ref,
                      mw1_ref, mb1_ref, mw2_ref, mb2_ref, wout_ref, whout_ref):
    d = d_ref[...]
    offs = jax.lax.broadcasted_iota(jnp.int32, (1, NG), 1).astype(jnp.float32) \
        * _SPACING
    de = jnp.exp(_COEFF * (d - offs) ** 2)
    c = c_ref[...]
    fh = fh_ref[...]
    for l in range(L):
        h1 = _ssp(jnp.dot(de, w1_ref[l], preferred_element_type=jnp.float32)
                  + b1_ref[l])
        w = jnp.dot(h1, w2_ref[l], preferred_element_type=jnp.float32) + b2_ref[l]
        wout_ref[l] = w * c
        hh = _ssp(jnp.dot(fh, mw1_ref[l], preferred_element_type=jnp.float32)
                  + mb1_ref[l])
        whout_ref[l] = jnp.dot(hh, mw2_ref[l],
                               preferred_element_type=jnp.float32) + mb2_ref[l]


def _edge_filter(d2, c2, fea_hull, params, block_e=1600):
    ne = E // block_e
    full = lambda shape: pl.BlockSpec(shape, lambda e: tuple(0 for _ in shape))
    in_specs = [
        pl.BlockSpec((block_e, 1), lambda e: (e, 0)),
        pl.BlockSpec((block_e, 1), lambda e: (e, 0)),
        pl.BlockSpec((block_e, 7), lambda e: (e, 0)),
        full((L, NG, NF)),
        full((L, 1, NF)),
        full((L, NF, NF)),
        full((L, 1, NF)),
        full((L, 7, NF)),
        full((L, 1, NF)),
        full((L, NF, NF)),
        full((L, 1, NF)),
    ]
    out_specs = [
        pl.BlockSpec((L, block_e, NF), lambda e: (0, e, 0)),
        pl.BlockSpec((L, block_e, NF), lambda e: (0, e, 0)),
    ]
    out_shape = [
        jax.ShapeDtypeStruct((L, E, NF), jnp.float32),
        jax.ShapeDtypeStruct((L, EH, NF), jnp.float32),
    ]
    return pl.pallas_call(
        _edge_filter_body,
        grid=(ne,),
        in_specs=in_specs,
        out_specs=out_specs,
        out_shape=out_shape,
    )(d2, c2, fea_hull,
      params["ue_mlp_W1"], params["ue_mlp_b1"].reshape(L, 1, NF),
      params["ue_mlp_W2"], params["ue_mlp_b2"].reshape(L, 1, NF),
      params["ue_mh_W1"], params["ue_mh_b1"].reshape(L, 1, NF),
      params["ue_mh_W2"], params["ue_mh_b2"].reshape(L, 1, NF))


_NC = 2
_NS = 16
_NW = _NC * _NS
_CH = 128
_NCHUNK = E // _CH
_NFULL = _NCHUNK // _NW


def _seg_body(zeros_hbm, vl_hbm, w_hbm, src_hbm, dst_hbm, out_hbm,
              sidx0, sidx1, didx0, didx1, gbuf0, gbuf1, wbuf,
              acc, gsem0, gsem1, isem0, isem1):
    c = lax.axis_index("c")
    s = lax.axis_index("s")
    wid = s * _NC + c
    sidxs = (sidx0, sidx1)
    didxs = (didx0, didx1)
    gbufs = (gbuf0, gbuf1)
    gsems = (gsem0, gsem1)
    isems = (isem0, isem1)

    rows = 624
    last = N - (_NS - 1) * rows
    base_r = pl.multiple_of(s * rows, 8)

    def zero_rows():
        @pl.when(s < _NS - 1)
        def _():
            pltpu.sync_copy(zeros_hbm.at[pl.ds(base_r, rows)],
                            acc.at[pl.ds(base_r, rows)])

        @pl.when(s == _NS - 1)
        def _():
            pltpu.sync_copy(zeros_hbm.at[pl.ds((_NS - 1) * rows, last)],
                            acc.at[pl.ds((_NS - 1) * rows, last)])

    def writeout():
        @pl.when(s < _NS - 1)
        def _():
            pltpu.sync_copy(acc.at[pl.ds(base_r, rows)],
                            out_hbm.at[c, pl.ds(base_r, rows)])

        @pl.when(s == _NS - 1)
        def _():
            pltpu.sync_copy(acc.at[pl.ds((_NS - 1) * rows, last)],
                            out_hbm.at[c, pl.ds((_NS - 1) * rows, last)])

    def run(vl_h, w_h, src_h, dst_h):
        lo = wid * _NCHUNK // _NW
        hi = (wid + 1) * _NCHUNK // _NW
        nch = hi - lo

        def i_descs(k, b):
            off = pl.multiple_of((lo + k) * _CH, 8)
            return (pltpu.make_async_copy(src_h.at[pl.ds(off, _CH)],
                                          sidxs[b], isems[b]),
                    pltpu.make_async_copy(dst_h.at[pl.ds(off, _CH)],
                                          didxs[b], isems[b]))

        def g_desc(k, b):
            return pltpu.make_async_copy(
                vl_h.at[sidxs[b]], gbufs[b], gsems[b])

        def issue_idx(k, b):
            d1, d2 = i_descs(k, b)
            d1.start()
            d2.start()

        def wait_idx(k, b):
            d1, d2 = i_descs(k, b)
            d1.wait()
            d2.wait()

        def process(k, b):
            off = pl.multiple_of((lo + k) * _CH, 8)
            pltpu.sync_copy(src_h.at[pl.ds(off, _CH)], sidxs[b])
            pltpu.sync_copy(dst_h.at[pl.ds(off, _CH)], didxs[b])
            pltpu.async_copy(vl_h.at[sidxs[b]], gbufs[b], gsems[b]).wait()
            pltpu.sync_copy(w_h.at[pl.ds(off, _CH)], wbuf)

            gb, wb = gbufs[b], wbuf

            def row(i, carry):
                for j in range(NF // 16):
                    gv = gb[i, pl.ds(j * 16, 16)]
                    wv = wb[i, pl.ds(j * 16, 16)]
                    gb[i, pl.ds(j * 16, 16)] = gv * wv
                return carry

            lax.fori_loop(0, _CH, row, 0, unroll=False)
            pltpu.sync_copy(gb, acc.at[didxs[b]], add=True)

        def pairs(j, carry):
            process(2 * j, 0)
            process(2 * j + 1, 1)
            return carry

        lax.fori_loop(0, _NFULL // 2, pairs, 0, unroll=False)

        @pl.when(nch > _NFULL)
        def _():
            process(jnp.int32(_NFULL), 0)

    zero_rows()
    plsc.subcore_barrier()
    run(vl_hbm, w_hbm, src_hbm, dst_hbm)
    plsc.subcore_barrier()
    writeout()


def _sc_layer(zeros, vl, w, src, dst):
    mesh = plsc.VectorSubcoreMesh(core_axis_name="c", subcore_axis_name="s",
                                  num_cores=_NC, num_subcores=_NS)
    f = pl.kernel(
        _seg_body,
        out_type=jax.ShapeDtypeStruct((_NC, N, NF), jnp.float32),
        mesh=mesh,
        scratch_types=[
            pltpu.VMEM((_CH,), jnp.int32),
            pltpu.VMEM((_CH,), jnp.int32),
            pltpu.VMEM((_CH,), jnp.int32),
            pltpu.VMEM((_CH,), jnp.int32),
            pltpu.VMEM((_CH, NF), jnp.float32),
            pltpu.VMEM((_CH, NF), jnp.float32),
            pltpu.VMEM((_CH, NF), jnp.float32),
            pltpu.VMEM_SHARED((N, NF), jnp.float32),
            pltpu.SemaphoreType.DMA,
            pltpu.SemaphoreType.DMA,
            pltpu.SemaphoreType.DMA,
            pltpu.SemaphoreType.DMA,
        ],
    )
    return f(zeros, vl, w, src, dst)


def kernel(z, dist, edge_index, fea_hull, edge_index_hull, batch, params):
    c_cut = 0.5 * (jnp.cos(dist * (pi / CUT)) + 1.0)
    w_all, wh_all = _edge_filter(dist.reshape(E, 1), c_cut.reshape(E, 1),
                                 fea_hull, params)

    v = params["emb_table"][z]
    src, dst = edge_index[0], edge_index[1]
    srch, dsth = edge_index_hull[0], edge_index_hull[1]

    zeros = jnp.zeros((N, NF), jnp.float32)
    for l in range(L):
        vl = v @ params["ue_lin"][l]
        vlh = v @ params["ue_lin_hull"][l]
        out_p = _sc_layer(zeros, vl, w_all[l], src, dst)
        outh_p = _sc_layer(zeros, vlh, wh_all[l], srch, dsth)
        out = out_p[0] + out_p[1]
        outh = outh_p[0] + outh_p[1]
        out = _ssp(out @ params["uv_lin1_W"][l] + params["uv_lin1_b"][l]) \
            @ params["uv_lin2_W"][l] + params["uv_lin2_b"][l]
        outh = _ssp(outh @ params["uv_lin1h_W"][l] + params["uv_lin1h_b"][l]) \
            @ params["uv_lin2h_W"][l] + params["uv_lin2h_b"][l]
        out = _ssp(jnp.concatenate([out, outh], axis=1) @ params["uv_cat_W"][l]
                   + params["uv_cat_b"][l])
        v = v + out

    h = _ssp(v @ params["uu_lin1_W"] + params["uu_lin1_b"])
    o = h @ params["uu_lin2_W"] + params["uu_lin2_b"]
    return jax.ops.segment_sum(o, batch, num_segments=NB)


# speedup vs baseline: 3.3822x; 1.3066x over previous
import functools
from math import pi, log

import numpy as np
import jax
import jax.numpy as jnp
from jax import lax
from jax.experimental import pallas as pl
from jax.experimental.pallas import tpu as pltpu
from jax.experimental.pallas import tpu_sc as plsc

N = 10000
E = 320000
EH = 320000
H = 128
NF = 128
NG = 25
NB = 512
L = 3
CUT = 5.0

_LOG2 = log(2.0)


def _ssp(x):
    return jnp.maximum(x, 0.0) + jnp.log(1.0 + jnp.exp(-jnp.abs(x))) - _LOG2


_SPACING = CUT / (NG - 1)
_COEFF = -0.5 / (_SPACING * _SPACING)
_OFFS = np.arange(NG, dtype=np.float32) * _SPACING
_M4 = np.zeros((4, NG), dtype=np.float32)
_M4[0] = _COEFF
_M4[1] = -2.0 * _COEFF * _OFFS
_M4[2] = _COEFF * _OFFS * _OFFS


def _edge_filter_body(d_ref, c_ref, fh_ref, w1_ref, b1_ref, w2_ref, b2_ref,
                      mw1_ref, mb1_ref, mw2_ref, mb2_ref, wout_ref, whout_ref):
    d = d_ref[...]
    offs = jax.lax.broadcasted_iota(jnp.int32, (1, NG), 1).astype(jnp.float32) \
        * _SPACING
    de = jnp.exp(_COEFF * (d - offs) ** 2)
    c = c_ref[...]
    fh = fh_ref[...]
    for l in range(L):
        h1 = _ssp(jnp.dot(de, w1_ref[l], preferred_element_type=jnp.float32)
                  + b1_ref[l])
        w = jnp.dot(h1, w2_ref[l], preferred_element_type=jnp.float32) + b2_ref[l]
        wout_ref[l] = w * c
        hh = _ssp(jnp.dot(fh, mw1_ref[l], preferred_element_type=jnp.float32)
                  + mb1_ref[l])
        whout_ref[l] = jnp.dot(hh, mw2_ref[l],
                               preferred_element_type=jnp.float32) + mb2_ref[l]


def _edge_filter(d2, c2, fea_hull, params, block_e=1600):
    ne = E // block_e
    full = lambda shape: pl.BlockSpec(shape, lambda e: tuple(0 for _ in shape))
    in_specs = [
        pl.BlockSpec((block_e, 1), lambda e: (e, 0)),
        pl.BlockSpec((block_e, 1), lambda e: (e, 0)),
        pl.BlockSpec((block_e, 7), lambda e: (e, 0)),
        full((L, NG, NF)),
        full((L, 1, NF)),
        full((L, NF, NF)),
        full((L, 1, NF)),
        full((L, 7, NF)),
        full((L, 1, NF)),
        full((L, NF, NF)),
        full((L, 1, NF)),
    ]
    out_specs = [
        pl.BlockSpec((L, block_e, NF), lambda e: (0, e, 0)),
        pl.BlockSpec((L, block_e, NF), lambda e: (0, e, 0)),
    ]
    out_shape = [
        jax.ShapeDtypeStruct((L, E, NF), jnp.float32),
        jax.ShapeDtypeStruct((L, EH, NF), jnp.float32),
    ]
    return pl.pallas_call(
        _edge_filter_body,
        grid=(ne,),
        in_specs=in_specs,
        out_specs=out_specs,
        out_shape=out_shape,
    )(d2, c2, fea_hull,
      params["ue_mlp_W1"], params["ue_mlp_b1"].reshape(L, 1, NF),
      params["ue_mlp_W2"], params["ue_mlp_b2"].reshape(L, 1, NF),
      params["ue_mh_W1"], params["ue_mh_b1"].reshape(L, 1, NF),
      params["ue_mh_W2"], params["ue_mh_b2"].reshape(L, 1, NF))


_NC = 2
_NS = 16
_NW = _NC * _NS
_CH = 128
_NCHUNK = E // _CH
_NFULL = _NCHUNK // _NW


def _seg_body(zeros_hbm, vl_hbm, w_hbm, src_hbm, dst_hbm, out_hbm,
              sidx0, sidx1, didx0, didx1, gbuf0, gbuf1, wbuf,
              acc, gsem0, gsem1, isem0, isem1, wsem):
    c = lax.axis_index("c")
    s = lax.axis_index("s")
    wid = s * _NC + c
    sidxs = (sidx0, sidx1)
    didxs = (didx0, didx1)
    gbufs = (gbuf0, gbuf1)
    gsems = (gsem0, gsem1)
    isems = (isem0, isem1)

    rows = 624
    last = N - (_NS - 1) * rows
    base_r = pl.multiple_of(s * rows, 8)

    def zero_rows():
        @pl.when(s < _NS - 1)
        def _():
            pltpu.sync_copy(zeros_hbm.at[pl.ds(base_r, rows)],
                            acc.at[pl.ds(base_r, rows)])

        @pl.when(s == _NS - 1)
        def _():
            pltpu.sync_copy(zeros_hbm.at[pl.ds((_NS - 1) * rows, last)],
                            acc.at[pl.ds((_NS - 1) * rows, last)])

    def writeout():
        @pl.when(s < _NS - 1)
        def _():
            pltpu.sync_copy(acc.at[pl.ds(base_r, rows)],
                            out_hbm.at[c, pl.ds(base_r, rows)])

        @pl.when(s == _NS - 1)
        def _():
            pltpu.sync_copy(acc.at[pl.ds((_NS - 1) * rows, last)],
                            out_hbm.at[c, pl.ds((_NS - 1) * rows, last)])

    def run(vl_h, w_h, src_h, dst_h):
        lo = wid * _NCHUNK // _NW
        hi = (wid + 1) * _NCHUNK // _NW
        nch = hi - lo

        def i_descs(k, b):
            off = pl.multiple_of((lo + k) * _CH, 8)
            return (pltpu.make_async_copy(src_h.at[pl.ds(off, _CH)],
                                          sidxs[b], isems[b]),
                    pltpu.make_async_copy(dst_h.at[pl.ds(off, _CH)],
                                          didxs[b], isems[b]))

        def g_desc(k, b):
            return pltpu.make_async_copy(
                vl_h.at[sidxs[b]], gbufs[b], gsems[b])

        def issue_idx(k, b):
            d1, d2 = i_descs(k, b)
            d1.start()
            d2.start()

        def wait_idx(k, b):
            d1, d2 = i_descs(k, b)
            d1.wait()
            d2.wait()

        def w_desc(k):
            off = pl.multiple_of((lo + k) * _CH, 8)
            return pltpu.make_async_copy(w_h.at[pl.ds(off, _CH)], wbuf, wsem)

        def process(k, b):
            w_desc(k).wait()
            g_desc(k, b).wait()

            gb, wb = gbufs[b], wbuf

            def row(i, carry):
                for j in range(NF // 16):
                    gv = gb[i, pl.ds(j * 16, 16)]
                    wv = wb[i, pl.ds(j * 16, 16)]
                    gb[i, pl.ds(j * 16, 16)] = gv * wv
                return carry

            lax.fori_loop(0, _CH, row, 0, unroll=False)

            @pl.when(k + 1 < nch)
            def _():
                w_desc(k + 1).start()

            pltpu.sync_copy(gb, acc.at[didxs[b]], add=True)

            @pl.when(k + 2 < nch)
            def _():
                issue_idx(k + 2, b)

            @pl.when(k + 1 < nch)
            def _():
                wait_idx(k + 1, b ^ 1)
                g_desc(k + 1, b ^ 1).start()

        issue_idx(0, 0)

        @pl.when(nch > 1)
        def _():
            issue_idx(1, 1)

        wait_idx(0, 0)
        g_desc(0, 0).start()
        w_desc(0).start()

        def pairs(j, carry):
            process(2 * j, 0)
            process(2 * j + 1, 1)
            return carry

        lax.fori_loop(0, _NFULL // 2, pairs, 0, unroll=False)

        @pl.when(nch > _NFULL)
        def _():
            process(jnp.int32(_NFULL), 0)

    zero_rows()
    plsc.subcore_barrier()
    run(vl_hbm, w_hbm, src_hbm, dst_hbm)
    plsc.subcore_barrier()
    writeout()


def _sc_layer(zeros, vl, w, src, dst):
    mesh = plsc.VectorSubcoreMesh(core_axis_name="c", subcore_axis_name="s",
                                  num_cores=_NC, num_subcores=_NS)
    f = pl.kernel(
        _seg_body,
        out_type=jax.ShapeDtypeStruct((_NC, N, NF), jnp.float32),
        mesh=mesh,
        scratch_types=[
            pltpu.VMEM((_CH,), jnp.int32),
            pltpu.VMEM((_CH,), jnp.int32),
            pltpu.VMEM((_CH,), jnp.int32),
            pltpu.VMEM((_CH,), jnp.int32),
            pltpu.VMEM((_CH, NF), jnp.float32),
            pltpu.VMEM((_CH, NF), jnp.float32),
            pltpu.VMEM((_CH, NF), jnp.float32),
            pltpu.VMEM_SHARED((N, NF), jnp.float32),
            pltpu.SemaphoreType.DMA,
            pltpu.SemaphoreType.DMA,
            pltpu.SemaphoreType.DMA,
            pltpu.SemaphoreType.DMA,
            pltpu.SemaphoreType.DMA,
        ],
    )
    return f(zeros, vl, w, src, dst)


def kernel(z, dist, edge_index, fea_hull, edge_index_hull, batch, params):
    c_cut = 0.5 * (jnp.cos(dist * (pi / CUT)) + 1.0)
    w_all, wh_all = _edge_filter(dist.reshape(E, 1), c_cut.reshape(E, 1),
                                 fea_hull, params)

    v = params["emb_table"][z]
    src, dst = edge_index[0], edge_index[1]
    srch, dsth = edge_index_hull[0], edge_index_hull[1]

    zeros = jnp.zeros((N, NF), jnp.float32)
    for l in range(L):
        vl = v @ params["ue_lin"][l]
        vlh = v @ params["ue_lin_hull"][l]
        out_p = _sc_layer(zeros, vl, w_all[l], src, dst)
        outh_p = _sc_layer(zeros, vlh, wh_all[l], srch, dsth)
        out = out_p[0] + out_p[1]
        outh = outh_p[0] + outh_p[1]
        out = _ssp(out @ params["uv_lin1_W"][l] + params["uv_lin1_b"][l]) \
            @ params["uv_lin2_W"][l] + params["uv_lin2_b"][l]
        outh = _ssp(outh @ params["uv_lin1h_W"][l] + params["uv_lin1h_b"][l]) \
            @ params["uv_lin2h_W"][l] + params["uv_lin2h_b"][l]
        out = _ssp(jnp.concatenate([out, outh], axis=1) @ params["uv_cat_W"][l]
                   + params["uv_cat_b"][l])
        v = v + out

    h = _ssp(v @ params["uu_lin1_W"] + params["uu_lin1_b"])
    o = h @ params["uu_lin2_W"] + params["uu_lin2_b"]
    return jax.ops.segment_sum(o, batch, num_segments=NB)


# speedup vs baseline: 4.2440x; 1.2548x over previous
import functools
from math import pi, log

import numpy as np
import jax
import jax.numpy as jnp
from jax import lax
from jax.experimental import pallas as pl
from jax.experimental.pallas import tpu as pltpu
from jax.experimental.pallas import tpu_sc as plsc

N = 10000
E = 320000
EH = 320000
H = 128
NF = 128
NG = 25
NB = 512
L = 3
CUT = 5.0

_LOG2 = log(2.0)


def _ssp(x):
    return jnp.maximum(x, 0.0) + jnp.log(1.0 + jnp.exp(-jnp.abs(x))) - _LOG2


_SPACING = CUT / (NG - 1)
_COEFF = -0.5 / (_SPACING * _SPACING)
_OFFS = np.arange(NG, dtype=np.float32) * _SPACING
_M4 = np.zeros((4, NG), dtype=np.float32)
_M4[0] = _COEFF
_M4[1] = -2.0 * _COEFF * _OFFS
_M4[2] = _COEFF * _OFFS * _OFFS


def _edge_filter_body(d_ref, c_ref, fh_ref, w1_ref, b1_ref, w2_ref, b2_ref,
                      mw1_ref, mb1_ref, mw2_ref, mb2_ref, wout_ref, whout_ref):
    d = d_ref[...]
    offs = jax.lax.broadcasted_iota(jnp.int32, (1, NG), 1).astype(jnp.float32) \
        * _SPACING
    de = jnp.exp(_COEFF * (d - offs) ** 2)
    c = c_ref[...]
    fh = fh_ref[...]
    h1 = _ssp(jnp.dot(de, w1_ref[0], preferred_element_type=jnp.float32)
              + b1_ref[0])
    w = jnp.dot(h1, w2_ref[0], preferred_element_type=jnp.float32) + b2_ref[0]
    wout_ref[...] = w * c
    hh = _ssp(jnp.dot(fh, mw1_ref[0], preferred_element_type=jnp.float32)
              + mb1_ref[0])
    whout_ref[...] = jnp.dot(hh, mw2_ref[0],
                             preferred_element_type=jnp.float32) + mb2_ref[0]


def _edge_filter(l, d2, c2, fea_hull, params, block_e=1600):
    ne = E // block_e
    full = lambda shape: pl.BlockSpec(shape, lambda e: tuple(0 for _ in shape))
    in_specs = [
        pl.BlockSpec((block_e, 1), lambda e: (e, 0)),
        pl.BlockSpec((block_e, 1), lambda e: (e, 0)),
        pl.BlockSpec((block_e, 7), lambda e: (e, 0)),
        full((1, NG, NF)),
        full((1, 1, NF)),
        full((1, NF, NF)),
        full((1, 1, NF)),
        full((1, 7, NF)),
        full((1, 1, NF)),
        full((1, NF, NF)),
        full((1, 1, NF)),
    ]
    out_specs = [
        pl.BlockSpec((block_e, NF), lambda e: (e, 0)),
        pl.BlockSpec((block_e, NF), lambda e: (e, 0)),
    ]
    out_shape = [
        jax.ShapeDtypeStruct((E, NF), jnp.float32),
        jax.ShapeDtypeStruct((EH, NF), jnp.float32),
    ]
    return pl.pallas_call(
        _edge_filter_body,
        grid=(ne,),
        in_specs=in_specs,
        out_specs=out_specs,
        out_shape=out_shape,
    )(d2, c2, fea_hull,
      params["ue_mlp_W1"][l:l + 1], params["ue_mlp_b1"][l].reshape(1, 1, NF),
      params["ue_mlp_W2"][l:l + 1], params["ue_mlp_b2"][l].reshape(1, 1, NF),
      params["ue_mh_W1"][l:l + 1], params["ue_mh_b1"][l].reshape(1, 1, NF),
      params["ue_mh_W2"][l:l + 1], params["ue_mh_b2"][l].reshape(1, 1, NF))


_NC = 2
_NS = 16
_NW = _NC * _NS
_CH = 128
_NCHUNK = E // _CH
_NFULL = _NCHUNK // _NW


def _seg_body(zeros_hbm, vl_hbm, w_hbm, src_hbm, dst_hbm, out_hbm,
              sidx0, sidx1, didx0, didx1, gbuf0, gbuf1, wbuf,
              acc, gsem0, gsem1, isem0, isem1, wsem):
    c = lax.axis_index("c")
    s = lax.axis_index("s")
    wid = s * _NC + c
    sidxs = (sidx0, sidx1)
    didxs = (didx0, didx1)
    gbufs = (gbuf0, gbuf1)
    gsems = (gsem0, gsem1)
    isems = (isem0, isem1)

    rows = 624
    last = N - (_NS - 1) * rows
    base_r = pl.multiple_of(s * rows, 8)

    def zero_rows():
        @pl.when(s < _NS - 1)
        def _():
            pltpu.sync_copy(zeros_hbm.at[pl.ds(base_r, rows)],
                            acc.at[pl.ds(base_r, rows)])

        @pl.when(s == _NS - 1)
        def _():
            pltpu.sync_copy(zeros_hbm.at[pl.ds((_NS - 1) * rows, last)],
                            acc.at[pl.ds((_NS - 1) * rows, last)])

    def writeout():
        @pl.when(s < _NS - 1)
        def _():
            pltpu.sync_copy(acc.at[pl.ds(base_r, rows)],
                            out_hbm.at[c, pl.ds(base_r, rows)])

        @pl.when(s == _NS - 1)
        def _():
            pltpu.sync_copy(acc.at[pl.ds((_NS - 1) * rows, last)],
                            out_hbm.at[c, pl.ds((_NS - 1) * rows, last)])

    def run(vl_h, w_h, src_h, dst_h):
        lo = wid * _NCHUNK // _NW
        hi = (wid + 1) * _NCHUNK // _NW
        nch = hi - lo

        def i_descs(k, b):
            off = pl.multiple_of((lo + k) * _CH, 8)
            return (pltpu.make_async_copy(src_h.at[pl.ds(off, _CH)],
                                          sidxs[b], isems[b]),
                    pltpu.make_async_copy(dst_h.at[pl.ds(off, _CH)],
                                          didxs[b], isems[b]))

        def g_desc(k, b):
            return pltpu.make_async_copy(
                vl_h.at[sidxs[b]], gbufs[b], gsems[b])

        def issue_idx(k, b):
            d1, d2 = i_descs(k, b)
            d1.start()
            d2.start()

        def wait_idx(k, b):
            d1, d2 = i_descs(k, b)
            d1.wait()
            d2.wait()

        def w_desc(k):
            off = pl.multiple_of((lo + k) * _CH, 8)
            return pltpu.make_async_copy(w_h.at[pl.ds(off, _CH)], wbuf, wsem)

        def process(k, b):
            w_desc(k).wait()
            g_desc(k, b).wait()

            gb, wb = gbufs[b], wbuf

            def row(i, carry):
                for j in range(NF // 16):
                    gv = gb[i, pl.ds(j * 16, 16)]
                    wv = wb[i, pl.ds(j * 16, 16)]
                    gb[i, pl.ds(j * 16, 16)] = gv * wv
                return carry

            lax.fori_loop(0, _CH, row, 0, unroll=False)

            @pl.when(k + 1 < nch)
            def _():
                w_desc(k + 1).start()

            pltpu.sync_copy(gb, acc.at[didxs[b]], add=True)

            @pl.when(k + 2 < nch)
            def _():
                issue_idx(k + 2, b)

            @pl.when(k + 1 < nch)
            def _():
                wait_idx(k + 1, b ^ 1)
                g_desc(k + 1, b ^ 1).start()

        issue_idx(0, 0)

        @pl.when(nch > 1)
        def _():
            issue_idx(1, 1)

        wait_idx(0, 0)
        g_desc(0, 0).start()
        w_desc(0).start()

        def pairs(j, carry):
            process(2 * j, 0)
            process(2 * j + 1, 1)
            return carry

        lax.fori_loop(0, _NFULL // 2, pairs, 0, unroll=False)

        @pl.when(nch > _NFULL)
        def _():
            process(jnp.int32(_NFULL), 0)

    zero_rows()
    plsc.subcore_barrier()
    run(vl_hbm, w_hbm, src_hbm, dst_hbm)
    plsc.subcore_barrier()
    writeout()


def _sc_layer(zeros, vl, w, src, dst):
    mesh = plsc.VectorSubcoreMesh(core_axis_name="c", subcore_axis_name="s",
                                  num_cores=_NC, num_subcores=_NS)
    f = pl.kernel(
        _seg_body,
        out_type=jax.ShapeDtypeStruct((_NC, N, NF), jnp.float32),
        mesh=mesh,
        scratch_types=[
            pltpu.VMEM((_CH,), jnp.int32),
            pltpu.VMEM((_CH,), jnp.int32),
            pltpu.VMEM((_CH,), jnp.int32),
            pltpu.VMEM((_CH,), jnp.int32),
            pltpu.VMEM((_CH, NF), jnp.float32),
            pltpu.VMEM((_CH, NF), jnp.float32),
            pltpu.VMEM((_CH, NF), jnp.float32),
            pltpu.VMEM_SHARED((N, NF), jnp.float32),
            pltpu.SemaphoreType.DMA,
            pltpu.SemaphoreType.DMA,
            pltpu.SemaphoreType.DMA,
            pltpu.SemaphoreType.DMA,
            pltpu.SemaphoreType.DMA,
        ],
    )
    return f(zeros, vl, w, src, dst)


def kernel(z, dist, edge_index, fea_hull, edge_index_hull, batch, params):
    c_cut = 0.5 * (jnp.cos(dist * (pi / CUT)) + 1.0)
    d2, c2 = dist.reshape(E, 1), c_cut.reshape(E, 1)

    v = params["emb_table"][z]
    src, dst = edge_index[0], edge_index[1]
    srch, dsth = edge_index_hull[0], edge_index_hull[1]

    zeros = jnp.zeros((N, NF), jnp.float32)
    for l in range(L):
        w_l, wh_l = _edge_filter(l, d2, c2, fea_hull, params)
        vl = v @ params["ue_lin"][l]
        vlh = v @ params["ue_lin_hull"][l]
        out_p = _sc_layer(zeros, vl, w_l, src, dst)
        outh_p = _sc_layer(zeros, vlh, wh_l, srch, dsth)
        out = out_p[0] + out_p[1]
        outh = outh_p[0] + outh_p[1]
        out = _ssp(out @ params["uv_lin1_W"][l] + params["uv_lin1_b"][l]) \
            @ params["uv_lin2_W"][l] + params["uv_lin2_b"][l]
        outh = _ssp(outh @ params["uv_lin1h_W"][l] + params["uv_lin1h_b"][l]) \
            @ params["uv_lin2h_W"][l] + params["uv_lin2h_b"][l]
        out = _ssp(jnp.concatenate([out, outh], axis=1) @ params["uv_cat_W"][l]
                   + params["uv_cat_b"][l])
        v = v + out

    h = _ssp(v @ params["uu_lin1_W"] + params["uu_lin1_b"])
    o = h @ params["uu_lin2_W"] + params["uu_lin2_b"]
    return jax.ops.segment_sum(o, batch, num_segments=NB)


# speedup vs baseline: 4.3372x; 1.0220x over previous
import functools
from math import pi, log

import numpy as np
import jax
import jax.numpy as jnp
from jax import lax
from jax.experimental import pallas as pl
from jax.experimental.pallas import tpu as pltpu
from jax.experimental.pallas import tpu_sc as plsc

N = 10000
E = 320000
EH = 320000
H = 128
NF = 128
NG = 25
NB = 512
L = 3
CUT = 5.0

_LOG2 = log(2.0)


def _ssp(x):
    return jnp.maximum(x, 0.0) + jnp.log(1.0 + jnp.exp(-jnp.abs(x))) - _LOG2


_SPACING = CUT / (NG - 1)
_COEFF = -0.5 / (_SPACING * _SPACING)
_OFFS = np.arange(NG, dtype=np.float32) * _SPACING
_M4 = np.zeros((4, NG), dtype=np.float32)
_M4[0] = _COEFF
_M4[1] = -2.0 * _COEFF * _OFFS
_M4[2] = _COEFF * _OFFS * _OFFS


def _edge_filter_body(d_ref, c_ref, fh_ref, w1_ref, b1_ref, w2_ref, b2_ref,
                      mw1_ref, mb1_ref, mw2_ref, mb2_ref, wout_ref, whout_ref):
    d = d_ref[...]
    offs = jax.lax.broadcasted_iota(jnp.int32, (1, NG), 1).astype(jnp.float32) \
        * _SPACING
    de = jnp.exp(_COEFF * (d - offs) ** 2)
    c = c_ref[...]
    fh = fh_ref[...]
    h1 = _ssp(jnp.dot(de, w1_ref[0], preferred_element_type=jnp.float32)
              + b1_ref[0])
    w = jnp.dot(h1, w2_ref[0], preferred_element_type=jnp.float32) + b2_ref[0]
    wout_ref[...] = w * c
    hh = _ssp(jnp.dot(fh, mw1_ref[0], preferred_element_type=jnp.float32)
              + mb1_ref[0])
    whout_ref[...] = jnp.dot(hh, mw2_ref[0],
                             preferred_element_type=jnp.float32) + mb2_ref[0]


def _edge_filter(l, d2, c2, fea_hull, params, block_e=1600):
    ne = E // block_e
    full = lambda shape: pl.BlockSpec(shape, lambda e: tuple(0 for _ in shape))
    in_specs = [
        pl.BlockSpec((block_e, 1), lambda e: (e, 0)),
        pl.BlockSpec((block_e, 1), lambda e: (e, 0)),
        pl.BlockSpec((block_e, 7), lambda e: (e, 0)),
        full((1, NG, NF)),
        full((1, 1, NF)),
        full((1, NF, NF)),
        full((1, 1, NF)),
        full((1, 7, NF)),
        full((1, 1, NF)),
        full((1, NF, NF)),
        full((1, 1, NF)),
    ]
    out_specs = [
        pl.BlockSpec((block_e, NF), lambda e: (e, 0)),
        pl.BlockSpec((block_e, NF), lambda e: (e, 0)),
    ]
    out_shape = [
        jax.ShapeDtypeStruct((E, NF), jnp.float32),
        jax.ShapeDtypeStruct((EH, NF), jnp.float32),
    ]
    return pl.pallas_call(
        _edge_filter_body,
        grid=(ne,),
        in_specs=in_specs,
        out_specs=out_specs,
        out_shape=out_shape,
    )(d2, c2, fea_hull,
      params["ue_mlp_W1"][l:l + 1], params["ue_mlp_b1"][l].reshape(1, 1, NF),
      params["ue_mlp_W2"][l:l + 1], params["ue_mlp_b2"][l].reshape(1, 1, NF),
      params["ue_mh_W1"][l:l + 1], params["ue_mh_b1"][l].reshape(1, 1, NF),
      params["ue_mh_W2"][l:l + 1], params["ue_mh_b2"][l].reshape(1, 1, NF))


_BN = 2000


def _init_body(z_ref, emb_ref, ue_ref, ueh_ref, v_ref, vl_ref, vlh_ref):
    z = z_ref[...]
    ids = jax.lax.broadcasted_iota(jnp.int32, (1, 100), 1)
    onehot = (z == ids).astype(jnp.float32)
    v = jnp.dot(onehot, emb_ref[...], preferred_element_type=jnp.float32)
    v_ref[...] = v
    vl_ref[...] = jnp.dot(v, ue_ref[...], preferred_element_type=jnp.float32)
    vlh_ref[...] = jnp.dot(v, ueh_ref[...], preferred_element_type=jnp.float32)


def _node_init(z2, params):
    full = lambda shape: pl.BlockSpec(shape, lambda i: tuple(0 for _ in shape))
    blk = lambda m: pl.BlockSpec((_BN, m), lambda i: (i, 0))
    return pl.pallas_call(
        _init_body,
        grid=(N // _BN,),
        in_specs=[blk(1), full((100, H)), full((H, NF)), full((H, NF))],
        out_specs=[blk(NF), blk(NF), blk(NF)],
        out_shape=[jax.ShapeDtypeStruct((N, NF), jnp.float32)] * 3,
    )(z2, params["emb_table"], params["ue_lin"][0], params["ue_lin_hull"][0])


def _node_mlp(out_p, outh_p, v, uv1, uv1b, uv2, uv2b, uv1h, uv1hb, uv2h,
              uv2hb, cat_t, cat_b, catb):
    s = out_p[0] + out_p[1]
    t = _ssp(jnp.dot(s, uv1, preferred_element_type=jnp.float32) + uv1b)
    t = jnp.dot(t, uv2, preferred_element_type=jnp.float32) + uv2b
    sh = outh_p[0] + outh_p[1]
    th = _ssp(jnp.dot(sh, uv1h, preferred_element_type=jnp.float32) + uv1hb)
    th = jnp.dot(th, uv2h, preferred_element_type=jnp.float32) + uv2hb
    u = _ssp(jnp.dot(t, cat_t, preferred_element_type=jnp.float32)
             + jnp.dot(th, cat_b, preferred_element_type=jnp.float32) + catb)
    return v + u


def _layer_body(op_ref, ohp_ref, v_ref, uv1_ref, uv1b_ref, uv2_ref, uv2b_ref,
                uv1h_ref, uv1hb_ref, uv2h_ref, uv2hb_ref, catt_ref, catb_ref,
                catbias_ref, ue_ref, ueh_ref, vn_ref, vl_ref, vlh_ref):
    vn = _node_mlp(op_ref[...], ohp_ref[...], v_ref[...],
                   uv1_ref[...], uv1b_ref[...], uv2_ref[...], uv2b_ref[...],
                   uv1h_ref[...], uv1hb_ref[...], uv2h_ref[...], uv2hb_ref[...],
                   catt_ref[...], catb_ref[...], catbias_ref[...])
    vn_ref[...] = vn
    vl_ref[...] = jnp.dot(vn, ue_ref[...], preferred_element_type=jnp.float32)
    vlh_ref[...] = jnp.dot(vn, ueh_ref[...], preferred_element_type=jnp.float32)


def _last_body(op_ref, ohp_ref, v_ref, uv1_ref, uv1b_ref, uv2_ref, uv2b_ref,
               uv1h_ref, uv1hb_ref, uv2h_ref, uv2hb_ref, catt_ref, catb_ref,
               catbias_ref, uu1_ref, uu1b_ref, uu2_ref, uu2b_ref, batch_ref,
               out_ref):
    vn = _node_mlp(op_ref[...], ohp_ref[...], v_ref[...],
                   uv1_ref[...], uv1b_ref[...], uv2_ref[...], uv2b_ref[...],
                   uv1h_ref[...], uv1hb_ref[...], uv2h_ref[...], uv2hb_ref[...],
                   catt_ref[...], catb_ref[...], catbias_ref[...])
    hh = _ssp(jnp.dot(vn, uu1_ref[...], preferred_element_type=jnp.float32)
              + uu1b_ref[...])
    o = jnp.dot(hh, uu2_ref[...], preferred_element_type=jnp.float32) \
        + uu2b_ref[...]
    ids = jax.lax.broadcasted_iota(jnp.int32, (1, NB), 1)
    onehot = (batch_ref[...] == ids).astype(jnp.float32)
    part = jax.lax.dot_general(o, onehot, (((0,), (0,)), ((), ())),
                               preferred_element_type=jnp.float32)

    @pl.when(pl.program_id(0) == 0)
    def _():
        out_ref[...] = jnp.zeros_like(out_ref)

    out_ref[...] += part


def _weight_specs():
    full = lambda shape: pl.BlockSpec(shape, lambda i: tuple(0 for _ in shape))
    return [
        full((NF, H)), full((1, H)), full((H, 64)), full((1, 64)),
        full((NF, H)), full((1, H)), full((H, 64)), full((1, 64)),
        full((64, H)), full((64, H)), full((1, H)),
    ]


def _layer_weights(params, l):
    catw = params["uv_cat_W"][l]
    return (params["uv_lin1_W"][l], params["uv_lin1_b"][l].reshape(1, H),
            params["uv_lin2_W"][l], params["uv_lin2_b"][l].reshape(1, 64),
            params["uv_lin1h_W"][l], params["uv_lin1h_b"][l].reshape(1, H),
            params["uv_lin2h_W"][l], params["uv_lin2h_b"][l].reshape(1, 64),
            catw[:64], catw[64:], params["uv_cat_b"][l].reshape(1, H))


def _node_layer(l, out_p, outh_p, v, params):
    full = lambda shape: pl.BlockSpec(shape, lambda i: tuple(0 for _ in shape))
    blkp = pl.BlockSpec((_NC, _BN, NF), lambda i: (0, i, 0))
    blk = lambda m: pl.BlockSpec((_BN, m), lambda i: (i, 0))
    in_specs = [blkp, blkp, blk(NF)] + _weight_specs() + \
        [full((H, NF)), full((H, NF))]
    return pl.pallas_call(
        _layer_body,
        grid=(N // _BN,),
        in_specs=in_specs,
        out_specs=[blk(NF), blk(NF), blk(NF)],
        out_shape=[jax.ShapeDtypeStruct((N, NF), jnp.float32)] * 3,
    )(out_p, outh_p, v, *_layer_weights(params, l),
      params["ue_lin"][l + 1], params["ue_lin_hull"][l + 1])


def _node_last(l, out_p, outh_p, v, batch2, params):
    full = lambda shape: pl.BlockSpec(shape, lambda i: tuple(0 for _ in shape))
    blkp = pl.BlockSpec((_NC, _BN, NF), lambda i: (0, i, 0))
    blk = lambda m: pl.BlockSpec((_BN, m), lambda i: (i, 0))
    in_specs = [blkp, blkp, blk(NF)] + _weight_specs() + [
        full((H, 64)), full((1, 64)), full((64, 1)), full((1, 1)), blk(1)]
    return pl.pallas_call(
        _last_body,
        grid=(N // _BN,),
        in_specs=in_specs,
        out_specs=pl.BlockSpec((1, NB), lambda i: (0, 0)),
        out_shape=jax.ShapeDtypeStruct((1, NB), jnp.float32),
    )(out_p, outh_p, v, *_layer_weights(params, l),
      params["uu_lin1_W"], params["uu_lin1_b"].reshape(1, 64),
      params["uu_lin2_W"], params["uu_lin2_b"].reshape(1, 1), batch2)


_NC = 2
_NS = 16
_NW = _NC * _NS
_CH = 128
_NCHUNK = E // _CH
_NFULL = _NCHUNK // _NW


def _seg_body(zeros_hbm, vl_hbm, w_hbm, src_hbm, dst_hbm, out_hbm,
              sidx0, sidx1, didx0, didx1, gbuf0, gbuf1, wbuf,
              acc, gsem0, gsem1, isem0, isem1, wsem):
    c = lax.axis_index("c")
    s = lax.axis_index("s")
    wid = s * _NC + c
    sidxs = (sidx0, sidx1)
    didxs = (didx0, didx1)
    gbufs = (gbuf0, gbuf1)
    gsems = (gsem0, gsem1)
    isems = (isem0, isem1)

    rows = 624
    last = N - (_NS - 1) * rows
    base_r = pl.multiple_of(s * rows, 8)

    def zero_rows():
        @pl.when(s < _NS - 1)
        def _():
            pltpu.sync_copy(zeros_hbm.at[pl.ds(base_r, rows)],
                            acc.at[pl.ds(base_r, rows)])

        @pl.when(s == _NS - 1)
        def _():
            pltpu.sync_copy(zeros_hbm.at[pl.ds((_NS - 1) * rows, last)],
                            acc.at[pl.ds((_NS - 1) * rows, last)])

    def writeout():
        @pl.when(s < _NS - 1)
        def _():
            pltpu.sync_copy(acc.at[pl.ds(base_r, rows)],
                            out_hbm.at[c, pl.ds(base_r, rows)])

        @pl.when(s == _NS - 1)
        def _():
            pltpu.sync_copy(acc.at[pl.ds((_NS - 1) * rows, last)],
                            out_hbm.at[c, pl.ds((_NS - 1) * rows, last)])

    def run(vl_h, w_h, src_h, dst_h):
        lo = wid * _NCHUNK // _NW
        hi = (wid + 1) * _NCHUNK // _NW
        nch = hi - lo

        def i_descs(k, b):
            off = pl.multiple_of((lo + k) * _CH, 8)
            return (pltpu.make_async_copy(src_h.at[pl.ds(off, _CH)],
                                          sidxs[b], isems[b]),
                    pltpu.make_async_copy(dst_h.at[pl.ds(off, _CH)],
                                          didxs[b], isems[b]))

        def g_desc(k, b):
            return pltpu.make_async_copy(
                vl_h.at[sidxs[b]], gbufs[b], gsems[b])

        def issue_idx(k, b):
            d1, d2 = i_descs(k, b)
            d1.start()
            d2.start()

        def wait_idx(k, b):
            d1, d2 = i_descs(k, b)
            d1.wait()
            d2.wait()

        def w_desc(k):
            off = pl.multiple_of((lo + k) * _CH, 8)
            return pltpu.make_async_copy(w_h.at[pl.ds(off, _CH)], wbuf, wsem)

        def process(k, b):
            w_desc(k).wait()
            g_desc(k, b).wait()

            gb, wb = gbufs[b], wbuf

            def row(i, carry):
                for j in range(NF // 16):
                    gv = gb[i, pl.ds(j * 16, 16)]
                    wv = wb[i, pl.ds(j * 16, 16)]
                    gb[i, pl.ds(j * 16, 16)] = gv * wv
                return carry

            lax.fori_loop(0, _CH, row, 0, unroll=False)

            @pl.when(k + 1 < nch)
            def _():
                w_desc(k + 1).start()

            pltpu.sync_copy(gb, acc.at[didxs[b]], add=True)

            @pl.when(k + 2 < nch)
            def _():
                issue_idx(k + 2, b)

            @pl.when(k + 1 < nch)
            def _():
                wait_idx(k + 1, b ^ 1)
                g_desc(k + 1, b ^ 1).start()

        issue_idx(0, 0)

        @pl.when(nch > 1)
        def _():
            issue_idx(1, 1)

        wait_idx(0, 0)
        g_desc(0, 0).start()
        w_desc(0).start()

        def pairs(j, carry):
            process(2 * j, 0)
            process(2 * j + 1, 1)
            return carry

        lax.fori_loop(0, _NFULL // 2, pairs, 0, unroll=False)

        @pl.when(nch > _NFULL)
        def _():
            process(jnp.int32(_NFULL), 0)

    zero_rows()
    plsc.subcore_barrier()
    run(vl_hbm, w_hbm, src_hbm, dst_hbm)
    plsc.subcore_barrier()
    writeout()


def _sc_layer(zeros, vl, w, src, dst):
    mesh = plsc.VectorSubcoreMesh(core_axis_name="c", subcore_axis_name="s",
                                  num_cores=_NC, num_subcores=_NS)
    f = pl.kernel(
        _seg_body,
        out_type=jax.ShapeDtypeStruct((_NC, N, NF), jnp.float32),
        mesh=mesh,
        scratch_types=[
            pltpu.VMEM((_CH,), jnp.int32),
            pltpu.VMEM((_CH,), jnp.int32),
            pltpu.VMEM((_CH,), jnp.int32),
            pltpu.VMEM((_CH,), jnp.int32),
            pltpu.VMEM((_CH, NF), jnp.float32),
            pltpu.VMEM((_CH, NF), jnp.float32),
            pltpu.VMEM((_CH, NF), jnp.float32),
            pltpu.VMEM_SHARED((N, NF), jnp.float32),
            pltpu.SemaphoreType.DMA,
            pltpu.SemaphoreType.DMA,
            pltpu.SemaphoreType.DMA,
            pltpu.SemaphoreType.DMA,
            pltpu.SemaphoreType.DMA,
        ],
    )
    return f(zeros, vl, w, src, dst)


def kernel(z, dist, edge_index, fea_hull, edge_index_hull, batch, params):
    c_cut = 0.5 * (jnp.cos(dist * (pi / CUT)) + 1.0)
    d2, c2 = dist.reshape(E, 1), c_cut.reshape(E, 1)

    src, dst = edge_index[0], edge_index[1]
    srch, dsth = edge_index_hull[0], edge_index_hull[1]

    v, vl, vlh = _node_init(z.reshape(N, 1), params)
    zeros = jnp.zeros((N, NF), jnp.float32)
    for l in range(L):
        w_l, wh_l = _edge_filter(l, d2, c2, fea_hull, params)
        out_p = _sc_layer(zeros, vl, w_l, src, dst)
        outh_p = _sc_layer(zeros, vlh, wh_l, srch, dsth)
        if l < L - 1:
            v, vl, vlh = _node_layer(l, out_p, outh_p, v, params)
        else:
            res = _node_last(l, out_p, outh_p, v, batch.reshape(N, 1), params)
    return res.reshape(NB, 1)
